# baseline (speedup 1.0000x reference)
import functools
import jax
import jax.numpy as jnp
from jax import lax
from jax.experimental import pallas as pl
from jax.experimental.pallas import tpu as pltpu
from jax.experimental.pallas import tpu_sc as plsc

DAMPING = 0.1
P = 2
NBUF = 2
LANES = 16


def _sc_body(x_hbm, pos_hbm, o_hbm, pos_t, ib0, ib1, ob0, ob1,
             psem, isem0, isem1, osem0, osem1):
    B, R, C, E = x_hbm.shape
    NCH = R // P
    w = lax.axis_index("s") * 2 + lax.axis_index("c")

    pltpu.make_async_copy(pos_hbm, pos_t, psem).start()
    pltpu.make_async_copy(pos_hbm, pos_t, psem).wait()

    ibufs = (ib0, ib1)
    obufs = (ob0, ob1)
    isems = (isem0, isem1)
    osems = (osem0, osem1)

    def in_copy(g, k):
        return pltpu.make_async_copy(
            x_hbm.at[w, pl.ds(g * P, P)], ibufs[k], isems[k])

    def out_copy(g, k):
        return pltpu.make_async_copy(
            obufs[k], o_hbm.at[w, pl.ds(g * P, P)], osems[k])

    def compute(g, k):
        ib, ob = ibufs[k], obufs[k]
        for p in range(P):
            r = g * P + p
            for eb in range(E // LANES):
                pv = pos_t[r, pl.ds(eb * LANES, LANES)] * DAMPING
                for c in range(C):
                    ob[p, c, pl.ds(eb * LANES, LANES)] = (
                        ib[p, c, pl.ds(eb * LANES, LANES)] + pv)

    for k in range(NBUF):
        in_copy(k, k).start()

    for k in range(NBUF):
        in_copy(k, k).wait()
        compute(k, k)
        out_copy(k, k).start()
        in_copy(k + NBUF, k).start()

    def mid(s, carry):
        base = s * NBUF
        for k in range(NBUF):
            g = base + k
            in_copy(g, k).wait()
            out_copy(g - NBUF, k).wait()
            compute(g, k)
            out_copy(g, k).start()
            in_copy(g + NBUF, k).start()
        return carry

    lax.fori_loop(1, NCH // NBUF - 1, mid, 0)

    for k in range(NBUF):
        g = NCH - NBUF + k
        in_copy(g, k).wait()
        out_copy(g - NBUF, k).wait()
        compute(g, k)
        out_copy(g, k).start()

    for k in range(NBUF):
        out_copy(NCH - NBUF + k, k).wait()


def kernel(x, pos_table):
    B, R, C, E = x.shape
    mesh = plsc.VectorSubcoreMesh(core_axis_name="c", subcore_axis_name="s")
    run = functools.partial(
        pl.kernel,
        mesh=mesh,
        out_type=jax.ShapeDtypeStruct(x.shape, x.dtype),
        scratch_types=[
            pltpu.VMEM((R, E), jnp.float32),
            pltpu.VMEM((P, C, E), jnp.float32),
            pltpu.VMEM((P, C, E), jnp.float32),
            pltpu.VMEM((P, C, E), jnp.float32),
            pltpu.VMEM((P, C, E), jnp.float32),
            pltpu.SemaphoreType.DMA,
            pltpu.SemaphoreType.DMA,
            pltpu.SemaphoreType.DMA,
            pltpu.SemaphoreType.DMA,
            pltpu.SemaphoreType.DMA,
        ],
    )(_sc_body)
    return run(x, pos_table)
